# Initial kernel scaffold; baseline (speedup 1.0000x reference)
import jax
import jax.numpy as jnp
from jax.experimental import pallas as pl


def kernel(x, edge_index, edge_type, weight0, w_comp0, self_loop0, weight1, w_comp1, self_loop1):
    raise NotImplementedError("write your pallas kernel here")


# speedup vs baseline: 4.5852x; 4.5852x over previous
import jax
import jax.numpy as jnp
from jax import lax
from jax.experimental import pallas as pl
from jax.experimental.pallas import tpu as pltpu
from jax.experimental.pallas import tpu_sc as plsc

N = 10000
E = 160000
INP = 128
EMB = 128
NB = 4
NR = 20

NC = 2
NS = 16
NW = NC * NS
CHUNK = 128
NCHUNK = 40
EPAD = NW * NCHUNK * CHUNK
NPAD = 10240
ROWS_PER_TILE = NPAD // NS
NBLK = 10
BLK = N // NBLK


def _project_body(h_ref, wb_ref, wc_ref, sl_ref, out_ref):
    r = pl.program_id(0)

    @pl.when(r < NR)
    def _():
        w = (wc_ref[r, 0] * wb_ref[0]
             + wc_ref[r, 1] * wb_ref[1]
             + wc_ref[r, 2] * wb_ref[2]
             + wc_ref[r, 3] * wb_ref[3])
        out_ref[...] = jnp.dot(h_ref[...], w, preferred_element_type=jnp.float32)

    @pl.when(r == NR)
    def _():
        out_ref[...] = jnp.dot(h_ref[...], sl_ref[...],
                               preferred_element_type=jnp.float32)


def _project(h, wb, wc, sl):
    return pl.pallas_call(
        _project_body,
        grid=(NR + 1, NBLK),
        in_specs=[
            pl.BlockSpec((BLK, INP), lambda r, n: (n, 0)),
            pl.BlockSpec((NB, INP, EMB), lambda r, n: (0, 0, 0)),
            pl.BlockSpec(memory_space=pltpu.SMEM),
            pl.BlockSpec((INP, EMB), lambda r, n: (0, 0)),
        ],
        out_specs=pl.BlockSpec((BLK, EMB), lambda r, n: (r * NBLK + n, 0)),
        out_shape=jax.ShapeDtypeStruct(((NR + 1) * N, EMB), jnp.float32),
    )(h, wb, wc, sl)


_MESH = plsc.VectorSubcoreMesh(core_axis_name="c", subcore_axis_name="s")
_NZ = ROWS_PER_TILE // CHUNK


def _zero_stripe(zrow, rows_v, agg_sh, base):
    pltpu.sync_copy(zrow, rows_v)
    for k in range(_NZ):
        pltpu.sync_copy(rows_v, agg_sh.at[pl.ds(base + k * CHUNK, CHUNK)])


def _write_stripe(agg_sh, rows_v, agg_out, c, base):
    for k in range(_NZ):
        row = base + k * CHUNK
        pltpu.sync_copy(agg_sh.at[pl.ds(row, CHUNK)], rows_v)
        pltpu.sync_copy(rows_v, agg_out.at[c, pl.ds(row, CHUNK)])


def _aggregate_body(table, gidx, dst, zrow, agg_out, agg_sh, gix_v, dix_v,
                    rows_v, sem):
    c = lax.axis_index("c")
    s = lax.axis_index("s")
    slab = s * NC + c
    base = s * ROWS_PER_TILE

    _zero_stripe(zrow, rows_v, agg_sh, base)
    plsc.subcore_barrier()

    def step(j, carry):
        pltpu.sync_copy(gidx.at[slab, j], gix_v)
        pltpu.sync_copy(dst.at[slab, j], dix_v)
        pltpu.async_copy(table.at[gix_v], rows_v, sem).wait()
        pltpu.sync_copy(rows_v, agg_sh.at[dix_v], add=True)
        return carry

    lax.fori_loop(0, NCHUNK, step, 0)
    plsc.subcore_barrier()
    _write_stripe(agg_sh, rows_v, agg_out, c, base)


_aggregate = pl.kernel(
    _aggregate_body,
    out_type=jax.ShapeDtypeStruct((NC, NPAD, EMB), jnp.float32),
    mesh=_MESH,
    scratch_types=[
        pltpu.VMEM_SHARED((NPAD, EMB), jnp.float32),
        pltpu.VMEM((CHUNK,), jnp.int32),
        pltpu.VMEM((CHUNK,), jnp.int32),
        pltpu.VMEM((CHUNK, EMB), jnp.float32),
        pltpu.SemaphoreType.DMA,
    ])


def _degree_body(dst, zrow, ones_row, deg_out, deg_sh, dix_v, rows_v):
    c = lax.axis_index("c")
    s = lax.axis_index("s")
    slab = s * NC + c
    base = s * ROWS_PER_TILE

    _zero_stripe(zrow, rows_v, deg_sh, base)
    pltpu.sync_copy(ones_row, rows_v)
    plsc.subcore_barrier()

    def step(j, carry):
        pltpu.sync_copy(dst.at[slab, j], dix_v)
        pltpu.sync_copy(rows_v, deg_sh.at[dix_v], add=True)
        return carry

    lax.fori_loop(0, NCHUNK, step, 0)
    plsc.subcore_barrier()
    _write_stripe(deg_sh, rows_v, deg_out, c, base)


_degree = pl.kernel(
    _degree_body,
    out_type=jax.ShapeDtypeStruct((NC, NPAD, EMB), jnp.float32),
    mesh=_MESH,
    scratch_types=[
        pltpu.VMEM_SHARED((NPAD, EMB), jnp.float32),
        pltpu.VMEM((CHUNK,), jnp.int32),
        pltpu.VMEM((CHUNK, EMB), jnp.float32),
    ])


def _combine_body(a_ref, m_ref, hrc_ref, out_ref):
    s = a_ref[0] + a_ref[1] + hrc_ref[...]
    m = m_ref[0, :, 0:1] + m_ref[1, :, 0:1]
    out_ref[...] = jnp.where(m > 0.0, jnp.maximum(s, 0.0), 0.0)


def _combine(agg_parts, deg_parts, hrc):
    return pl.pallas_call(
        _combine_body,
        grid=(NBLK,),
        in_specs=[
            pl.BlockSpec((NC, BLK, EMB), lambda n: (0, n, 0)),
            pl.BlockSpec((NC, BLK, EMB), lambda n: (0, n, 0)),
            pl.BlockSpec((BLK, EMB), lambda n: (NR * NBLK + n, 0)),
        ],
        out_specs=pl.BlockSpec((BLK, EMB), lambda n: (n, 0)),
        out_shape=jax.ShapeDtypeStruct((N, EMB), jnp.float32),
    )(agg_parts, deg_parts, hrc)


def kernel(x, edge_index, edge_type, weight0, w_comp0, self_loop0,
           weight1, w_comp1, self_loop1):
    src = edge_index[0]
    dst = edge_index[1]

    pad = EPAD - E
    gidx = edge_type * N + src
    pad_g = (jnp.arange(pad, dtype=jnp.int32) * 64) % (NR * N)
    pad_d = N + (jnp.arange(pad, dtype=jnp.int32) % (NPAD - N))
    gidx_p = jnp.concatenate([gidx, pad_g]).reshape(NW, NCHUNK, CHUNK)
    dst_p = jnp.concatenate([dst, pad_d]).reshape(NW, NCHUNK, CHUNK)

    zrow = jnp.zeros((CHUNK, EMB), jnp.float32)
    ones_row = jnp.ones((CHUNK, EMB), jnp.float32)

    deg = _degree(dst_p, zrow, ones_row)

    hrc0 = _project(x, weight0, w_comp0, self_loop0)
    agg0 = _aggregate(hrc0, gidx_p, dst_p, zrow)
    h1 = _combine(agg0, deg, hrc0)

    hrc1 = _project(h1, weight1, w_comp1, self_loop1)
    agg1 = _aggregate(hrc1, gidx_p, dst_p, zrow)
    h2 = _combine(agg1, deg, hrc1)
    return h2


# speedup vs baseline: 5.1890x; 1.1317x over previous
import jax
import jax.numpy as jnp
from jax import lax
from jax.experimental import pallas as pl
from jax.experimental.pallas import tpu as pltpu
from jax.experimental.pallas import tpu_sc as plsc

N = 10000
E = 160000
INP = 128
EMB = 128
NB = 4
NR = 20

NC = 2
NS = 16
NW = NC * NS
CHUNK = 128
NCHUNK = 40
EPAD = NW * NCHUNK * CHUNK
NPAD = 10240
ROWS_PER_TILE = NPAD // NS
NBLK = 10
BLK = N // NBLK


def _project_body(h_ref, wb_ref, wc_ref, sl_ref, out_ref):
    r = pl.program_id(0)

    @pl.when(r < NR)
    def _():
        w = (wc_ref[r, 0] * wb_ref[0]
             + wc_ref[r, 1] * wb_ref[1]
             + wc_ref[r, 2] * wb_ref[2]
             + wc_ref[r, 3] * wb_ref[3])
        out_ref[...] = jnp.dot(h_ref[...], w, preferred_element_type=jnp.float32)

    @pl.when(r == NR)
    def _():
        out_ref[...] = jnp.dot(h_ref[...], sl_ref[...],
                               preferred_element_type=jnp.float32)


def _project(h, wb, wc, sl):
    return pl.pallas_call(
        _project_body,
        grid=(NR + 1, NBLK),
        in_specs=[
            pl.BlockSpec((BLK, INP), lambda r, n: (n, 0)),
            pl.BlockSpec((NB, INP, EMB), lambda r, n: (0, 0, 0)),
            pl.BlockSpec(memory_space=pltpu.SMEM),
            pl.BlockSpec((INP, EMB), lambda r, n: (0, 0)),
        ],
        out_specs=pl.BlockSpec((BLK, EMB), lambda r, n: (r * NBLK + n, 0)),
        out_shape=jax.ShapeDtypeStruct(((NR + 1) * N, EMB), jnp.float32),
    )(h, wb, wc, sl)


_MESH = plsc.VectorSubcoreMesh(core_axis_name="c", subcore_axis_name="s")
_NZ = ROWS_PER_TILE // CHUNK


def _zero_stripe(zrow, rows_v, agg_sh, base):
    pltpu.sync_copy(zrow, rows_v)
    for k in range(_NZ):
        pltpu.sync_copy(rows_v, agg_sh.at[pl.ds(base + k * CHUNK, CHUNK)])


def _write_stripe(agg_sh, rows_v, agg_out, c, base):
    for k in range(_NZ):
        row = base + k * CHUNK
        pltpu.sync_copy(agg_sh.at[pl.ds(row, CHUNK)], rows_v)
        pltpu.sync_copy(rows_v, agg_out.at[c, pl.ds(row, CHUNK)])


def _aggregate_body(table, gidx, dst, zrow, agg_out, agg_sh, gix_v, dix_v,
                    rows_v, sem, gix2_v, dix2_v, rows2_v, sem2):
    c = lax.axis_index("c")
    s = lax.axis_index("s")
    slab = s * NC + c
    base = s * ROWS_PER_TILE

    _zero_stripe(zrow, rows_v, agg_sh, base)
    plsc.subcore_barrier()

    def step(j, carry):
        a = 2 * j
        pltpu.sync_copy(gidx.at[slab, a], gix_v)
        pltpu.sync_copy(dst.at[slab, a], dix_v)
        ca = pltpu.async_copy(table.at[gix_v], rows_v, sem)
        pltpu.sync_copy(gidx.at[slab, a + 1], gix2_v)
        pltpu.sync_copy(dst.at[slab, a + 1], dix2_v)
        cb = pltpu.async_copy(table.at[gix2_v], rows2_v, sem2)
        ca.wait()
        pltpu.sync_copy(rows_v, agg_sh.at[dix_v], add=True)
        cb.wait()
        pltpu.sync_copy(rows2_v, agg_sh.at[dix2_v], add=True)
        return carry

    lax.fori_loop(0, NCHUNK // 2, step, 0)
    plsc.subcore_barrier()
    _write_stripe(agg_sh, rows_v, agg_out, c, base)


_aggregate = pl.kernel(
    _aggregate_body,
    out_type=jax.ShapeDtypeStruct((NC, NPAD, EMB), jnp.float32),
    mesh=_MESH,
    scratch_types=[
        pltpu.VMEM_SHARED((NPAD, EMB), jnp.float32),
        pltpu.VMEM((CHUNK,), jnp.int32),
        pltpu.VMEM((CHUNK,), jnp.int32),
        pltpu.VMEM((CHUNK, EMB), jnp.float32),
        pltpu.SemaphoreType.DMA,
        pltpu.VMEM((CHUNK,), jnp.int32),
        pltpu.VMEM((CHUNK,), jnp.int32),
        pltpu.VMEM((CHUNK, EMB), jnp.float32),
        pltpu.SemaphoreType.DMA,
    ])


def _degree_body(dst, zrow, ones_row, deg_out, deg_sh, dix_v, rows_v):
    c = lax.axis_index("c")
    s = lax.axis_index("s")
    slab = s * NC + c
    base = s * ROWS_PER_TILE

    _zero_stripe(zrow, rows_v, deg_sh, base)
    pltpu.sync_copy(ones_row, rows_v)
    plsc.subcore_barrier()

    def step(j, carry):
        pltpu.sync_copy(dst.at[slab, j], dix_v)
        pltpu.sync_copy(rows_v, deg_sh.at[dix_v], add=True)
        return carry

    lax.fori_loop(0, NCHUNK, step, 0)
    plsc.subcore_barrier()
    _write_stripe(deg_sh, rows_v, deg_out, c, base)


_degree = pl.kernel(
    _degree_body,
    out_type=jax.ShapeDtypeStruct((NC, NPAD, EMB), jnp.float32),
    mesh=_MESH,
    scratch_types=[
        pltpu.VMEM_SHARED((NPAD, EMB), jnp.float32),
        pltpu.VMEM((CHUNK,), jnp.int32),
        pltpu.VMEM((CHUNK, EMB), jnp.float32),
    ])


def _combine_body(a_ref, m_ref, hrc_ref, out_ref):
    s = a_ref[0] + a_ref[1] + hrc_ref[...]
    m = m_ref[0, :, 0:1] + m_ref[1, :, 0:1]
    out_ref[...] = jnp.where(m > 0.0, jnp.maximum(s, 0.0), 0.0)


def _combine(agg_parts, deg_parts, hrc):
    return pl.pallas_call(
        _combine_body,
        grid=(NBLK,),
        in_specs=[
            pl.BlockSpec((NC, BLK, EMB), lambda n: (0, n, 0)),
            pl.BlockSpec((NC, BLK, EMB), lambda n: (0, n, 0)),
            pl.BlockSpec((BLK, EMB), lambda n: (NR * NBLK + n, 0)),
        ],
        out_specs=pl.BlockSpec((BLK, EMB), lambda n: (n, 0)),
        out_shape=jax.ShapeDtypeStruct((N, EMB), jnp.float32),
    )(agg_parts, deg_parts, hrc)


def kernel(x, edge_index, edge_type, weight0, w_comp0, self_loop0,
           weight1, w_comp1, self_loop1):
    src = edge_index[0]
    dst = edge_index[1]

    pad = EPAD - E
    gidx = edge_type * N + src
    pad_g = (jnp.arange(pad, dtype=jnp.int32) * 64) % (NR * N)
    pad_d = N + (jnp.arange(pad, dtype=jnp.int32) % (NPAD - N))
    gidx_p = jnp.concatenate([gidx, pad_g]).reshape(NW, NCHUNK, CHUNK)
    dst_p = jnp.concatenate([dst, pad_d]).reshape(NW, NCHUNK, CHUNK)

    zrow = jnp.zeros((CHUNK, EMB), jnp.float32)
    ones_row = jnp.ones((CHUNK, EMB), jnp.float32)

    deg = _degree(dst_p, zrow, ones_row)

    hrc0 = _project(x, weight0, w_comp0, self_loop0)
    agg0 = _aggregate(hrc0, gidx_p, dst_p, zrow)
    h1 = _combine(agg0, deg, hrc0)

    hrc1 = _project(h1, weight1, w_comp1, self_loop1)
    agg1 = _aggregate(hrc1, gidx_p, dst_p, zrow)
    h2 = _combine(agg1, deg, hrc1)
    return h2
